# baseline (speedup 1.0000x reference)
import functools

import jax
import jax.numpy as jnp
from jax import lax
from jax.experimental import pallas as pl
from jax.experimental.pallas import tpu as pltpu
from jax.experimental.pallas import tpu_sc as plsc

LANES = 16


@functools.lru_cache(maxsize=None)
def _build(b: int, nk: int, d: int, v: int, nw: int, nc: int):
    bpw = b // nw
    dt = d // 8
    nct = nk * dt
    mesh = plsc.VectorSubcoreMesh(core_axis_name="c", subcore_axis_name="s")

    @functools.partial(
        pl.kernel,
        mesh=mesh,
        compiler_params=pltpu.CompilerParams(
            use_tc_tiling_on_sc=False, needs_layout_passes=False
        ),
        out_type=jax.ShapeDtypeStruct((nct, nw, 8, bpw), jnp.float32),
        scratch_types=[
            pltpu.VMEM((nk, bpw), jnp.int32),
            pltpu.VMEM((2, bpw, d), jnp.float32),
            pltpu.VMEM((2, dt, 8, bpw), jnp.float32),
            pltpu.SemaphoreType.DMA((2,)),
            pltpu.SemaphoreType.DMA((2,)),
        ],
    )
    def gather_k(idx_hbm, table_hbm, out_hbm, idx_v, rows_v, tile_v, sem_g, sem_s):
        wid = lax.axis_index("s") * nc + lax.axis_index("c")
        pltpu.sync_copy(idx_hbm.at[:, wid], idx_v)

        lvecs = [lax.iota(jnp.int32, LANES) + LANES * h
                 for h in range(bpw // LANES)]

        def fire_gather(k, s2):
            return pltpu.async_copy(
                table_hbm.at[idx_v.at[k]], rows_v.at[s2], sem_g.at[s2]
            )

        def wait_gather(k, s2):
            pltpu.make_async_copy(
                table_hbm.at[idx_v.at[k]], rows_v.at[s2], sem_g.at[s2]
            ).wait()

        def fire_store(k, s2):
            return pltpu.async_copy(
                tile_v.at[s2], out_hbm.at[pl.ds(dt * k, dt), wid], sem_s.at[s2]
            )

        def wait_store(s2):
            pltpu.make_async_copy(
                tile_v.at[s2], out_hbm.at[pl.ds(0, dt), wid], sem_s.at[s2]
            ).wait()

        def transpose(s2):
            rows = rows_v.at[s2]
            for di in range(dt):
                for sub in range(8):
                    dcol = jnp.full((LANES,), 8 * di + sub, jnp.int32)
                    for h in range(bpw // LANES):
                        vec = plsc.load_gather(rows, [lvecs[h], dcol])
                        tile_v[s2, di, sub, pl.ds(LANES * h, LANES)] = vec

        g0 = fire_gather(0, 0)
        g1 = fire_gather(1, 1)
        g0.wait()
        transpose(0)
        fire_store(0, 0)
        fire_gather(2, 0)
        g1.wait()
        transpose(1)
        fire_store(1, 1)
        fire_gather(3, 1)

        def body(i, carry):
            for s2 in range(2):
                k = 2 * i + s2
                wait_gather(k, s2)
                wait_store(s2)
                transpose(s2)
                fire_store(k, s2)
                fire_gather(k + 2, s2)
            return carry

        lax.fori_loop(1, nk // 2 - 1, body, 0)
        for s2 in range(2):
            k = nk - 2 + s2
            wait_gather(k, s2)
            wait_store(s2)
            transpose(s2)
            fire_store(k, s2)
        for s2 in range(2):
            wait_store(s2)

    return gather_k


def kernel(query_indices, seq_indices, table):
    b = query_indices.shape[0]
    v, d = table.shape
    idx = jnp.concatenate(
        [query_indices.astype(jnp.int32), seq_indices.astype(jnp.int32)], axis=1
    )
    nk = idx.shape[1]
    info = plsc.get_sparse_core_info()
    nc, ns = info.num_cores, info.num_subcores
    nw = nc * ns
    bpw = b // nw
    idx3t = idx.T.reshape(nk, nw, bpw)
    out4 = _build(b, nk, d, v, nw, nc)(idx3t, table)
    return out4.transpose(1, 3, 0, 2).reshape(b, nk * d)


# speedup vs baseline: 1.0712x; 1.0712x over previous
import functools

import jax
import jax.numpy as jnp
from jax import lax
from jax.experimental import pallas as pl
from jax.experimental.pallas import tpu as pltpu
from jax.experimental.pallas import tpu_sc as plsc

LANES = 16


@functools.lru_cache(maxsize=None)
def _build(b: int, nk: int, d: int, v: int, nw: int, nc: int):
    bpw = b // nw
    dt = d // 8
    tile_f = 8 * bpw
    mesh = plsc.VectorSubcoreMesh(core_axis_name="c", subcore_axis_name="s")

    @functools.partial(
        pl.kernel,
        mesh=mesh,
        compiler_params=pltpu.CompilerParams(
            use_tc_tiling_on_sc=False, needs_layout_passes=False
        ),
        out_type=jax.ShapeDtypeStruct((nk * dt * nw * tile_f,), jnp.float32),
        scratch_types=[
            pltpu.VMEM((nk, bpw), jnp.int32),
            pltpu.VMEM((2, bpw, d), jnp.float32),
            pltpu.VMEM((2, dt * tile_f), jnp.float32),
            pltpu.SemaphoreType.DMA((2,)),
            pltpu.SemaphoreType.DMA((2,)),
        ],
    )
    def gather_k(idx_hbm, table_hbm, out_hbm, idx_v, rows_v, tile_v, sem_g, sem_s):
        wid = lax.axis_index("s") * nc + lax.axis_index("c")
        pltpu.sync_copy(idx_hbm.at[:, wid], idx_v)

        lvecs = [lax.iota(jnp.int32, LANES) + LANES * h
                 for h in range(bpw // LANES)]

        def fire_gather(k, s2):
            return pltpu.async_copy(
                table_hbm.at[idx_v.at[k]], rows_v.at[s2], sem_g.at[s2]
            )

        def wait_gather(k, s2):
            pltpu.make_async_copy(
                table_hbm.at[idx_v.at[k]], rows_v.at[s2], sem_g.at[s2]
            ).wait()

        def fire_store(k, s2):
            for di in range(dt):
                pltpu.async_copy(
                    tile_v.at[s2, pl.ds(di * tile_f, tile_f)],
                    out_hbm.at[pl.ds(((dt * k + di) * nw + wid) * tile_f, tile_f)],
                    sem_s.at[s2],
                )

        def wait_store(s2):
            for di in range(dt):
                pltpu.make_async_copy(
                    tile_v.at[s2, pl.ds(di * tile_f, tile_f)],
                    out_hbm.at[pl.ds(0, tile_f)],
                    sem_s.at[s2],
                ).wait()

        def transpose(s2):
            rows = rows_v.at[s2]

            def tbody(dd, carry):
                dcol = jnp.full((LANES,), dd, jnp.int32)
                for h in range(bpw // LANES):
                    vec = plsc.load_gather(rows, [lvecs[h], dcol])
                    tile_v[s2, pl.ds(dd * bpw + LANES * h, LANES)] = vec
                return carry

            lax.fori_loop(0, d, tbody, 0, unroll=4)

        g0 = fire_gather(0, 0)
        g1 = fire_gather(1, 1)
        g0.wait()
        transpose(0)
        fire_store(0, 0)
        fire_gather(2, 0)
        g1.wait()
        transpose(1)
        fire_store(1, 1)
        fire_gather(3, 1)

        def body(i, carry):
            for s2 in range(2):
                k = 2 * i + s2
                wait_gather(k, s2)
                wait_store(s2)
                transpose(s2)
                fire_store(k, s2)
                fire_gather(k + 2, s2)
            return carry

        lax.fori_loop(1, nk // 2 - 1, body, 0)
        for s2 in range(2):
            k = nk - 2 + s2
            wait_gather(k, s2)
            wait_store(s2)
            transpose(s2)
            fire_store(k, s2)
        for s2 in range(2):
            wait_store(s2)

    return gather_k


def kernel(query_indices, seq_indices, table):
    b = query_indices.shape[0]
    v, d = table.shape
    idx = jnp.concatenate(
        [query_indices.astype(jnp.int32), seq_indices.astype(jnp.int32)], axis=1
    )
    nk = idx.shape[1]
    info = plsc.get_sparse_core_info()
    nc, ns = info.num_cores, info.num_subcores
    nw = nc * ns
    bpw = b // nw
    idx3t = idx.T.reshape(nk, nw, bpw)
    out1 = _build(b, nk, d, v, nw, nc)(idx3t, table)
    out4 = out1.reshape(nk * d // 8, nw, 8, bpw)
    return out4.transpose(1, 3, 0, 2).reshape(b, nk * d)


# speedup vs baseline: 1.6259x; 1.5178x over previous
import functools

import jax
import jax.numpy as jnp
from jax import lax
from jax.experimental import pallas as pl
from jax.experimental.pallas import tpu as pltpu
from jax.experimental.pallas import tpu_sc as plsc

CH = 128
NBUF = 4


@functools.lru_cache(maxsize=None)
def _build(nchunk: int, d: int, v: int, nw: int, nc: int):
    n_main = (nchunk // NBUF) * NBUF
    tail = nchunk - n_main
    mesh = plsc.VectorSubcoreMesh(core_axis_name="c", subcore_axis_name="s")

    @functools.partial(
        pl.kernel,
        mesh=mesh,
        compiler_params=pltpu.CompilerParams(use_tc_tiling_on_sc=False),
        out_type=jax.ShapeDtypeStruct((nw, nchunk, CH, d), jnp.float32),
        scratch_types=[
            pltpu.VMEM((nchunk, CH), jnp.int32),
            pltpu.VMEM((NBUF, CH, d), jnp.float32),
            pltpu.SemaphoreType.DMA((NBUF,)),
            pltpu.SemaphoreType.DMA((NBUF,)),
        ],
    )
    def gather_k(idx_hbm, table_hbm, out_hbm, idx_v, rows_v, sem_g, sem_s):
        wid = lax.axis_index("s") * nc + lax.axis_index("c")
        pltpu.sync_copy(idx_hbm.at[wid], idx_v)

        def fire_gather(ch, b):
            return pltpu.async_copy(
                table_hbm.at[idx_v.at[ch]], rows_v.at[b], sem_g.at[b]
            )

        def fire_store(ch, b):
            return pltpu.async_copy(
                rows_v.at[b], out_hbm.at[wid, ch], sem_s.at[b]
            )

        def wait_store(b):
            pltpu.make_async_copy(
                rows_v.at[b], out_hbm.at[wid, 0], sem_s.at[b]
            ).wait()

        def step(base, nch, first):
            gathers = []
            for b in range(nch):
                if not first:
                    wait_store(b)
                gathers.append(fire_gather(base + b, b))
            for b in range(nch):
                gathers[b].wait()
                fire_store(base + b, b)

        step(0, NBUF, True)

        def body(i, carry):
            step(i * NBUF, NBUF, False)
            return carry

        lax.fori_loop(1, n_main // NBUF, body, 0)
        if tail:
            step(n_main, tail, False)
        for b in range(NBUF):
            wait_store(b)

    return gather_k


def kernel(query_indices, seq_indices, table):
    b = query_indices.shape[0]
    v, d = table.shape
    idx = jnp.concatenate(
        [query_indices.astype(jnp.int32), seq_indices.astype(jnp.int32)], axis=1
    )
    total = idx.size
    info = plsc.get_sparse_core_info()
    nc, ns = info.num_cores, info.num_subcores
    nw = nc * ns
    assert total % (nw * CH) == 0
    nchunk = total // (nw * CH)
    idx3 = idx.reshape(nw, nchunk, CH)
    out = _build(nchunk, d, v, nw, nc)(idx3, table)
    return out.reshape(b, -1)


# speedup vs baseline: 1.6517x; 1.0158x over previous
import functools

import jax
import jax.numpy as jnp
from jax import lax
from jax.experimental import pallas as pl
from jax.experimental.pallas import tpu as pltpu
from jax.experimental.pallas import tpu_sc as plsc

CH = 128
NBUF = 8


@functools.lru_cache(maxsize=None)
def _build(nchunk: int, d: int, v: int, nw: int, nc: int):
    n_main = (nchunk // NBUF) * NBUF
    tail = nchunk - n_main
    mesh = plsc.VectorSubcoreMesh(core_axis_name="c", subcore_axis_name="s")

    @functools.partial(
        pl.kernel,
        mesh=mesh,
        compiler_params=pltpu.CompilerParams(use_tc_tiling_on_sc=False),
        out_type=jax.ShapeDtypeStruct((nw, nchunk, CH, d), jnp.float32),
        scratch_types=[
            pltpu.VMEM((nchunk, CH), jnp.int32),
            pltpu.VMEM((NBUF, CH, d), jnp.float32),
            pltpu.SemaphoreType.DMA((NBUF,)),
            pltpu.SemaphoreType.DMA((NBUF,)),
        ],
    )
    def gather_k(idx_hbm, table_hbm, out_hbm, idx_v, rows_v, sem_g, sem_s):
        wid = lax.axis_index("s") * nc + lax.axis_index("c")
        pltpu.sync_copy(idx_hbm.at[wid], idx_v)

        def fire_gather(ch, b):
            return pltpu.async_copy(
                table_hbm.at[idx_v.at[ch]], rows_v.at[b], sem_g.at[b]
            )

        def fire_store(ch, b):
            return pltpu.async_copy(
                rows_v.at[b], out_hbm.at[wid, ch], sem_s.at[b]
            )

        def wait_store(b):
            pltpu.make_async_copy(
                rows_v.at[b], out_hbm.at[wid, 0], sem_s.at[b]
            ).wait()

        def step(base, nch, first):
            gathers = []
            for b in range(nch):
                if not first:
                    wait_store(b)
                gathers.append(fire_gather(base + b, b))
            for b in range(nch):
                gathers[b].wait()
                fire_store(base + b, b)

        step(0, NBUF, True)

        def body(i, carry):
            step(i * NBUF, NBUF, False)
            return carry

        lax.fori_loop(1, n_main // NBUF, body, 0)
        if tail:
            step(n_main, tail, False)
        for b in range(NBUF):
            wait_store(b)

    return gather_k


def kernel(query_indices, seq_indices, table):
    b = query_indices.shape[0]
    v, d = table.shape
    idx = jnp.concatenate(
        [query_indices.astype(jnp.int32), seq_indices.astype(jnp.int32)], axis=1
    )
    total = idx.size
    info = plsc.get_sparse_core_info()
    nc, ns = info.num_cores, info.num_subcores
    nw = nc * ns
    assert total % (nw * CH) == 0
    nchunk = total // (nw * CH)
    idx3 = idx.reshape(nw, nchunk, CH)
    out = _build(nchunk, d, v, nw, nc)(idx3, table)
    return out.reshape(b, -1)
